# Initial kernel scaffold; baseline (speedup 1.0000x reference)
import jax
import jax.numpy as jnp
from jax.experimental import pallas as pl


def kernel(outputs, targets, anchors):
    raise NotImplementedError("write your pallas kernel here")


# speedup vs baseline: 626.8597x; 626.8597x over previous
import jax
import jax.numpy as jnp
from jax.experimental import pallas as pl
from jax.experimental.pallas import tpu as pltpu

_NUM_CLASSES = 20
_IGNORE_THRESH = 0.75
_OBJ_SCALE = 5.0
_FS = 26
_F2 = _FS * _FS
_B = 16
_M = 20
_A = 5


def _iou_terms(ax1, ax2, ay1, ay2, aarea, bx1, bx2, by1, by2, barea):
    tlx = jnp.maximum(ax1, bx1)
    brx = jnp.minimum(ax2, bx2)
    tly = jnp.maximum(ay1, by1)
    bry = jnp.minimum(ay2, by2)
    iw = jnp.maximum(brx - tlx, 0.0)
    ih = jnp.maximum(bry - tly, 0.0)
    inter = iw * ih
    return inter / (aarea + barea - inter + 1e-10)


def _yolo_loss_kernel(o_ref, tx_ref, ty_ref, tw_ref, th_ref, tc_ref,
                      txt_ref, tyt_ref, twt_ref, tht_ref, tct_ref,
                      anc_ref, out_ref):
    f32 = jnp.float32
    i32 = jnp.int32

    tx = tx_ref[...]
    ty = ty_ref[...]
    tw = tw_ref[...]
    th = th_ref[...]
    tcl = tc_ref[...]
    valid = (tx + ty + tw + th + tcl) > 0.0

    gx = tx * 26.0
    gy = ty * 26.0
    gw = tw * 26.0
    gh = th * 26.0
    gx1 = gx - gw / 2.0
    gx2 = gx + gw / 2.0
    gy1 = gy - gh / 2.0
    gy2 = gy + gh / 2.0
    garea = gw * gh

    cxf = jnp.clip(jnp.floor(gx1), 0.0, 25.0)
    cyf = jnp.clip(jnp.floor(gy1), 0.0, 25.0)
    cxi = cxf.astype(i32)
    cyi = cyf.astype(i32)
    cell_idx = cyi * 26 + cxi

    best = None
    for a in range(_A):
        wa = anc_ref[a, 0] * 26.0
        ha = anc_ref[a, 1] * 26.0
        axc = cxf + 0.5
        ayc = cyf + 0.5
        iou_a = _iou_terms(axc - wa / 2.0, axc + wa / 2.0,
                           ayc - ha / 2.0, ayc + ha / 2.0, wa * ha,
                           gx1, gx2, gy1, gy2, garea)
        if best is None:
            best = iou_a
            bidx = jnp.zeros_like(cell_idx)
            wsel = jnp.zeros_like(gx) + wa
            hsel = jnp.zeros_like(gx) + ha
        else:
            upd = iou_a > best
            best = jnp.where(upd, iou_a, best)
            bidx = jnp.where(upd, a, bidx)
            wsel = jnp.where(upd, wa, wsel)
            hsel = jnp.where(upd, ha, hsel)

    dx = gx - (cxf + 0.5)
    dy = gy - (cyf + 0.5)
    dw = gw / wsel
    dh = gh / hsel

    key = cell_idx * 5 + bidx
    lane = jax.lax.broadcasted_iota(i32, (_B, _M), 1)
    ow = jnp.zeros((_B, _M), jnp.bool_)
    for nj in range(1, _M):
        ow = ow | ((key == key[:, nj:nj + 1])
                   & valid[:, nj:nj + 1] & (lane < nj))
    w_assign = jnp.where(valid & ~ow, 1.0, 0.0)

    def cat5(x):
        return jnp.concatenate([x] * 5, axis=0)

    ox = jnp.concatenate([o_ref[:, a * 25 + 0, :] for a in range(_A)], axis=0)
    oy = jnp.concatenate([o_ref[:, a * 25 + 1, :] for a in range(_A)], axis=0)
    ow_ = jnp.concatenate([o_ref[:, a * 25 + 2, :] for a in range(_A)], axis=0)
    oh_ = jnp.concatenate([o_ref[:, a * 25 + 3, :] for a in range(_A)], axis=0)
    oc_ = jnp.concatenate([o_ref[:, a * 25 + 4, :] for a in range(_A)], axis=0)

    R = _A * _B
    lane_p = jax.lax.broadcasted_iota(i32, (R, _F2), 1)
    xs_f = (lane_p % 26).astype(f32)
    ys_f = (lane_p // 26).astype(f32)
    row_a = jax.lax.broadcasted_iota(i32, (R, _F2), 0) // _B
    wa80 = jnp.zeros((R, _F2), f32)
    ha80 = jnp.zeros((R, _F2), f32)
    for a in range(_A):
        wa80 = jnp.where(row_a == a, anc_ref[a, 0] * 26.0, wa80)
        ha80 = jnp.where(row_a == a, anc_ref[a, 1] * 26.0, ha80)

    px = jax.nn.sigmoid(ox) + xs_f
    py = jax.nn.sigmoid(oy) + ys_f
    pw = jnp.exp(ow_) * wa80
    ph = jnp.exp(oh_) * ha80
    pax1 = px - pw / 2.0
    pax2 = px + pw / 2.0
    pay1 = py - ph / 2.0
    pay2 = py + ph / 2.0
    parea = pw * ph

    gx80_1 = cat5(gx1)
    gx80_2 = cat5(gx2)
    gy80_1 = cat5(gy1)
    gy80_2 = cat5(gy2)
    garea80 = cat5(garea)
    valid80 = cat5(valid)

    maxiou = jnp.full((R, _F2), -1.0, f32)
    for m in range(_M):
        iou = _iou_terms(pax1, pax2, pay1, pay2, parea,
                         gx80_1[:, m:m + 1], gx80_2[:, m:m + 1],
                         gy80_1[:, m:m + 1], gy80_2[:, m:m + 1],
                         garea80[:, m:m + 1])
        maxiou = jnp.maximum(maxiou,
                             jnp.where(valid80[:, m:m + 1], iou, -1.0))

    sigc = jax.nn.sigmoid(oc_)
    base = jnp.where(maxiou >= _IGNORE_THRESH, 0.0, sigc)
    dense_sum = jnp.sum(base * base)

    vsumT = (txt_ref[...] + tyt_ref[...] + twt_ref[...]
             + tht_ref[...] + tct_ref[...])
    gxT = txt_ref[...] * 26.0
    gyT = tyt_ref[...] * 26.0
    gwT = twt_ref[...] * 26.0
    ghT = tht_ref[...] * 26.0

    iota_cells = jax.lax.broadcasted_iota(i32, (_F2, _M), 0)
    iota_cls = jax.lax.broadcasted_iota(i32, (_NUM_CLASSES, _M), 0)

    acc_sq = jnp.zeros((), f32)
    acc_cls = jnp.zeros((), f32)
    for b in range(_B):
        onehot = (iota_cells == cell_idx[b:b + 1, :]).astype(f32)
        g = jax.lax.dot_general(o_ref[b], onehot,
                                (((1,), (0,)), ((), ())),
                                preferred_element_type=f32)
        arow = bidx[b:b + 1, :]
        sel = jnp.zeros((25, _M), f32)
        for a in range(_A):
            sel = sel + jnp.where(arow == a, 1.0, 0.0) * g[a * 25:(a + 1) * 25, :]

        sxr = jax.nn.sigmoid(sel[0:1, :])
        syr = jax.nn.sigmoid(sel[1:2, :])
        ewr = jnp.exp(sel[2:3, :])
        ehr = jnp.exp(sel[3:4, :])
        box_term = ((sxr - dx[b:b + 1, :]) ** 2 + (syr - dy[b:b + 1, :]) ** 2
                    + (ewr - dw[b:b + 1, :]) ** 2 + (ehr - dh[b:b + 1, :]) ** 2)

        pbx = sxr + cxf[b:b + 1, :]
        pby = syr + cyf[b:b + 1, :]
        pbw = ewr * wsel[b:b + 1, :]
        pbh = ehr * hsel[b:b + 1, :]
        bgx = gxT[:, b:b + 1]
        bgy = gyT[:, b:b + 1]
        bgw = gwT[:, b:b + 1]
        bgh = ghT[:, b:b + 1]
        iou_pm = _iou_terms(pbx - pbw / 2.0, pbx + pbw / 2.0,
                            pby - pbh / 2.0, pby + pbh / 2.0, pbw * pbh,
                            bgx - bgw / 2.0, bgx + bgw / 2.0,
                            bgy - bgh / 2.0, bgy + bgh / 2.0,
                            bgw * bgh)
        iou_pm = jnp.where(vsumT[:, b:b + 1] > 0.0, iou_pm, -1.0)
        it_row = jnp.max(iou_pm, axis=0, keepdims=True)

        sigcb = jax.nn.sigmoid(sel[4:5, :])
        base_pos = jnp.where(it_row >= _IGNORE_THRESH, 0.0, sigcb)
        iou_corr = (_OBJ_SCALE * (sigcb - it_row)) ** 2 - base_pos * base_pos

        logits = sel[5:25, :]
        lmax = jnp.max(logits, axis=0, keepdims=True)
        e = jnp.exp(logits - lmax)
        probs = e / jnp.sum(e, axis=0, keepdims=True)
        logp = jnp.log(probs + 1e-10)
        ci = jnp.clip(tcl[b:b + 1, :].astype(i32), 0, _NUM_CLASSES - 1)
        picked = jnp.sum(jnp.where(iota_cls == ci, logp, 0.0),
                         axis=0, keepdims=True)

        wb = w_assign[b:b + 1, :]
        acc_sq = acc_sq + jnp.sum(wb * (box_term + iou_corr))
        acc_cls = acc_cls + jnp.sum(wb * picked)

    out_ref[0, 0] = (dense_sum + acc_sq) / (2.0 * _B) - acc_cls / _B


def _loss(outputs, targets, anchors):
    o = outputs.reshape(_B, _A * (5 + _NUM_CLASSES), _F2)
    cols = [targets[:, :, k] for k in range(5)]
    colsT = [jnp.transpose(c) for c in cols]
    vspec = pl.BlockSpec(memory_space=pltpu.VMEM)
    out = pl.pallas_call(
        _yolo_loss_kernel,
        out_shape=jax.ShapeDtypeStruct((1, 1), jnp.float32),
        in_specs=[vspec] * 11 + [pl.BlockSpec(memory_space=pltpu.SMEM)],
        out_specs=pl.BlockSpec(memory_space=pltpu.SMEM),
    )(o, *cols, *colsT, anchors)
    return out[0, 0]


def kernel(outputs, targets, anchors):
    return _loss(outputs, targets, anchors)
